# Initial kernel scaffold; baseline (speedup 1.0000x reference)
import jax
import jax.numpy as jnp
from jax.experimental import pallas as pl


def kernel(feature, edge_index, W, b):
    raise NotImplementedError("write your pallas kernel here")


# speedup vs baseline: 7.3550x; 7.3550x over previous
import functools

import jax
import jax.numpy as jnp
from jax import lax
from jax.experimental import pallas as pl
from jax.experimental.pallas import tpu as pltpu
from jax.experimental.pallas import tpu_sc as plsc

N = 10000
E = 320000
D = 128
K = 8
NC = 2
NS = 16
L = 16
DH = D // NC

NP = 10240
RPT = NP // NS
RCH = 128
NRCH = RPT // RCH
CH = 128
EPT = 20480
NCHUNK = EPT // CH
EPAD = NS * EPT
FAKE = NP - 1
GRP = 2


def _rsqrt_newton(x):
    i = plsc.bitcast(x, jnp.int32)
    i = jnp.int32(0x5F3759DF) - (i >> 1)
    y = plsc.bitcast(i, jnp.float32)
    for _ in range(3):
        y = y * (1.5 - 0.5 * x * y * y)
    return y


def _sc_body(feat_hbm, src_hbm, dst_hbm, hs_hbm, ytab_hbm,
             src_v, dst_v, buf_a, buf_b,
             deg_v, deg2_v, dis_v, gsem, ssem, z_s, degp_s):
    c = lax.axis_index("c")
    s = lax.axis_index("s")
    bufs = [buf_a, buf_b]
    zero16 = jnp.zeros((L,), jnp.float32)
    one16 = jnp.ones((L,), jnp.float32)

    pltpu.sync_copy(src_hbm.at[c, s], src_v)
    pltpu.sync_copy(dst_hbm.at[s], dst_v)

    def _zero_deg(i, _):
        deg_v[pl.ds(i * L, L)] = zero16
        return 0
    lax.fori_loop(0, NP // L, _zero_deg, 0)

    def _count(t, _):
        j = t // (CH // L)
        i = t - j * (CH // L)
        idx = dst_v[j, pl.ds(i * L, L)]
        plsc.addupdate_scatter(deg_v, [idx], one16)
        return 0
    lax.fori_loop(0, NCHUNK * (CH // L), _count, 0)

    pltpu.sync_copy(deg_v, degp_s.at[s])
    plsc.subcore_barrier()
    for r in range(NS):
        pltpu.sync_copy(degp_s.at[r, pl.ds(s * RPT, RPT)], deg2_v.at[r])

    def _dis_chunk(i, _):
        acc = zero16
        for r in range(NS):
            acc = acc + deg2_v[r, pl.ds(i * L, L)]
        dis_v[pl.ds(i * L, L)] = _rsqrt_newton(acc + 1.0)
        return 0
    lax.fori_loop(0, RPT // L, _dis_chunk, 0)

    def _scale_rows(buf, q):
        def _grp(g, _):
            dv = dis_v[pl.ds(q * RCH + g * L, L)]
            for rr in range(L):
                sv = jnp.full((L,), dv[rr], jnp.float32)
                r = g * L + rr
                for jj in range(DH // L):
                    buf[r, pl.ds(jj * L, L)] = buf[r, pl.ds(jj * L, L)] * sv
            return 0
        lax.fori_loop(0, RCH // L, _grp, 0)

    def _init_chunk(q, _):
        row0 = s * RPT + q * RCH
        pltpu.sync_copy(feat_hbm.at[c, pl.ds(row0, RCH)], buf_a)
        pltpu.sync_copy(buf_a, hs_hbm.at[0, c, pl.ds(row0, RCH)])
        _scale_rows(buf_a, q)
        pltpu.sync_copy(buf_a, ytab_hbm.at[pl.ds(c * NP + row0, RCH)])
        pltpu.sync_copy(buf_a, z_s.at[pl.ds(row0, RCH)])
        return 0
    lax.fori_loop(0, NRCH, _init_chunk, 0)

    def _hop(k, _):
        plsc.subcore_barrier()

        def _group(g, _):
            j0 = g * GRP
            gets = [
                pltpu.async_copy(
                    ytab_hbm.at[src_v.at[j0 + t]], bufs[t], gsem)
                for t in range(GRP)
            ]
            puts = []
            for t in range(GRP):
                gets[t].wait()
                puts.append(pltpu.async_copy(
                    bufs[t], z_s.at[dst_v.at[j0 + t]], ssem, add=True))
            for p in puts:
                p.wait()
            return 0
        lax.fori_loop(0, NCHUNK // GRP, _group, 0)

        plsc.subcore_barrier()

        def _rescale_chunk(q, _):
            row0 = s * RPT + q * RCH
            pltpu.sync_copy(z_s.at[pl.ds(row0, RCH)], buf_a)
            _scale_rows(buf_a, q)
            pltpu.sync_copy(buf_a, hs_hbm.at[k + 1, c, pl.ds(row0, RCH)])
            _scale_rows(buf_a, q)
            pltpu.sync_copy(buf_a, ytab_hbm.at[pl.ds(c * NP + row0, RCH)])
            pltpu.sync_copy(buf_a, z_s.at[pl.ds(row0, RCH)])
            return 0
        lax.fori_loop(0, NRCH, _rescale_chunk, 0)
        return 0
    lax.fori_loop(0, K, _hop, 0)


@jax.jit
def _sc_propagate(feat_pad, src4, dst3):
    mesh = plsc.VectorSubcoreMesh(core_axis_name="c", subcore_axis_name="s")
    f = pl.kernel(
        _sc_body,
        out_type=(
            jax.ShapeDtypeStruct((K + 1, NC, NP, DH), jnp.float32),
            jax.ShapeDtypeStruct((NC * NP, DH), jnp.float32),
        ),
        mesh=mesh,
        scratch_types=[
            pltpu.VMEM((NCHUNK, CH), jnp.int32),
            pltpu.VMEM((NCHUNK, CH), jnp.int32),
            pltpu.VMEM((RCH, DH), jnp.float32),
            pltpu.VMEM((RCH, DH), jnp.float32),
            pltpu.VMEM((NP,), jnp.float32),
            pltpu.VMEM((NS, RPT), jnp.float32),
            pltpu.VMEM((RPT,), jnp.float32),
            pltpu.SemaphoreType.DMA,
            pltpu.SemaphoreType.DMA,
            pltpu.VMEM_SHARED((NP, DH), jnp.float32),
            pltpu.VMEM_SHARED((NS, NP), jnp.float32),
        ],
        compiler_params=pltpu.CompilerParams(
            use_tc_tiling_on_sc=False, needs_layout_passes=False),
    )
    return f(feat_pad, src4, dst3)


def _mm_body(hs_ref, w_ref, b_ref, o_ref):
    tt = pl.program_id(1)

    @pl.when(tt == 0)
    def _():
        o_ref[...] = jnp.broadcast_to(b_ref[...], o_ref.shape)

    o_ref[...] += jnp.dot(hs_ref[0, 0], w_ref[0, 0],
                          preferred_element_type=jnp.float32)


@jax.jit
def _tc_linear(hs, W4, b2):
    BM = 1024
    return pl.pallas_call(
        _mm_body,
        grid=(NP // BM, (K + 1) * NC),
        in_specs=[
            pl.BlockSpec((1, 1, BM, DH), lambda r, t: (t // NC, t % NC, r, 0)),
            pl.BlockSpec((1, 1, DH, D), lambda r, t: (t // NC, t % NC, 0, 0)),
            pl.BlockSpec((1, D), lambda r, t: (0, 0)),
        ],
        out_specs=pl.BlockSpec((BM, D), lambda r, t: (r, 0)),
        out_shape=jax.ShapeDtypeStruct((NP, D), jnp.float32),
        compiler_params=pltpu.CompilerParams(
            dimension_semantics=("parallel", "arbitrary")),
    )(hs, W4, b2)


def kernel(feature, edge_index, W, b):
    src = edge_index[0].astype(jnp.int32)
    dst = edge_index[1].astype(jnp.int32)
    pad = EPAD - E
    fake = jnp.full((pad,), FAKE, jnp.int32)
    src = jnp.concatenate([src, fake]).reshape(NS, NCHUNK, CH)
    dst = jnp.concatenate([dst, fake]).reshape(NS, NCHUNK, CH)
    src4 = jnp.stack([src, src + NP])
    feat_pad = jnp.pad(feature, ((0, NP - N), (0, 0)))
    feat_cs = jnp.stack([feat_pad[:, :DH], feat_pad[:, DH:]])

    hs, _ = _sc_propagate(feat_cs, src4, dst)
    out = _tc_linear(hs, W.reshape(K + 1, NC, DH, D), b.reshape(1, D))
    return out[:N]


# speedup vs baseline: 14.0238x; 1.9067x over previous
import functools

import jax
import jax.numpy as jnp
from jax import lax
from jax.experimental import pallas as pl
from jax.experimental.pallas import tpu as pltpu
from jax.experimental.pallas import tpu_sc as plsc

N = 10000
E = 320000
D = 128
K = 8
NC = 2
NS = 16
L = 16
DH = D // NC

NP = 10240
RPT = NP // NS
RCH = 128
NRCH = RPT // RCH
CH = 128
EPT = 20480
NCHUNK = EPT // CH
EPAD = NS * EPT
FAKE = NP - 1
GRP = 2
NG = NCHUNK // GRP


def _rsqrt_newton(x):
    i = plsc.bitcast(x, jnp.int32)
    i = jnp.int32(0x5F3759DF) - (i >> 1)
    y = plsc.bitcast(i, jnp.float32)
    for _ in range(3):
        y = y * (1.5 - 0.5 * x * y * y)
    return y


def _sc_body(feat_hbm, srci_hbm, dsti_hbm, hs_hbm,
             buf_a, buf_b, isrc_v, idst_v, deg_v, rbuf_v, dis_v,
             gsem, ssem, isem0, isem1, ytab_s, z_s, degp_s):
    c = lax.axis_index("c")
    s = lax.axis_index("s")
    bufs = [buf_a, buf_b]
    isems = [isem0, isem1]
    zero16 = jnp.zeros((L,), jnp.float32)
    one16 = jnp.ones((L,), jnp.float32)

    def _zero_deg(i, _):
        deg_v[pl.ds(i * L, L)] = zero16
        return 0
    lax.fori_loop(0, NP // L, _zero_deg, 0)

    pltpu.async_copy(dsti_hbm.at[s, pl.ds(0, GRP)], idst_v.at[0], isem0)

    def _deg_groups(g2, _):
        for b in range(2):
            g = 2 * g2 + b
            pltpu.make_async_copy(
                dsti_hbm.at[s, pl.ds(0, GRP)], idst_v.at[b], isems[b]).wait()

            @pl.when(g + 1 < NG)
            def _():
                pltpu.async_copy(
                    dsti_hbm.at[s, pl.ds((g + 1) * GRP, GRP)],
                    idst_v.at[1 - b], isems[1 - b])

            def _count(i, _):
                t = i // (CH // L)
                ii = i - t * (CH // L)
                idx = idst_v[b, t, pl.ds(ii * L, L)]
                plsc.addupdate_scatter(deg_v, [idx], one16)
                return 0
            lax.fori_loop(0, GRP * (CH // L), _count, 0)
        return 0
    lax.fori_loop(0, NG // 2, _deg_groups, 0)

    pltpu.sync_copy(deg_v, degp_s.at[s])
    plsc.subcore_barrier()

    def _zero_dis(i, _):
        dis_v[pl.ds(i * L, L)] = zero16
        return 0
    lax.fori_loop(0, RPT // L, _zero_dis, 0)
    for r in range(NS):
        pltpu.sync_copy(degp_s.at[r, pl.ds(s * RPT, RPT)], rbuf_v)

        def _acc(i, _):
            dis_v[pl.ds(i * L, L)] = (dis_v[pl.ds(i * L, L)]
                                      + rbuf_v[pl.ds(i * L, L)])
            return 0
        lax.fori_loop(0, RPT // L, _acc, 0)

    def _dis_chunk(i, _):
        dis_v[pl.ds(i * L, L)] = _rsqrt_newton(
            dis_v[pl.ds(i * L, L)] + 1.0)
        return 0
    lax.fori_loop(0, RPT // L, _dis_chunk, 0)

    def _scale_rows(buf, q):
        def _grp(g, _):
            dv = dis_v[pl.ds(q * RCH + g * L, L)]
            for rr in range(L):
                sv = jnp.full((L,), dv[rr], jnp.float32)
                r = g * L + rr
                for jj in range(DH // L):
                    buf[r, pl.ds(jj * L, L)] = buf[r, pl.ds(jj * L, L)] * sv
            return 0
        lax.fori_loop(0, RCH // L, _grp, 0)

    def _init_chunk(q, _):
        row0 = s * RPT + q * RCH
        pltpu.sync_copy(feat_hbm.at[c, pl.ds(row0, RCH)], buf_a)
        pltpu.sync_copy(buf_a, hs_hbm.at[0, c, pl.ds(row0, RCH)])
        _scale_rows(buf_a, q)
        pltpu.sync_copy(buf_a, ytab_s.at[pl.ds(row0, RCH)])
        pltpu.sync_copy(buf_a, z_s.at[pl.ds(row0, RCH)])
        return 0
    lax.fori_loop(0, NRCH, _init_chunk, 0)

    def _hop(k, _):
        plsc.subcore_barrier()

        pltpu.async_copy(srci_hbm.at[s, pl.ds(0, GRP)], isrc_v.at[0], isem0)
        pltpu.async_copy(dsti_hbm.at[s, pl.ds(0, GRP)], idst_v.at[0], isem0)

        def _group2(g2, _):
            for b in range(2):
                g = 2 * g2 + b
                pltpu.make_async_copy(
                    srci_hbm.at[s, pl.ds(0, GRP)], isrc_v.at[b],
                    isems[b]).wait()
                pltpu.make_async_copy(
                    dsti_hbm.at[s, pl.ds(0, GRP)], idst_v.at[b],
                    isems[b]).wait()

                @pl.when(g + 1 < NG)
                def _():
                    off = (g + 1) * GRP
                    pltpu.async_copy(
                        srci_hbm.at[s, pl.ds(off, GRP)], isrc_v.at[1 - b],
                        isems[1 - b])
                    pltpu.async_copy(
                        dsti_hbm.at[s, pl.ds(off, GRP)], idst_v.at[1 - b],
                        isems[1 - b])

                gets = [
                    pltpu.async_copy(
                        ytab_s.at[isrc_v.at[b, t]], bufs[t], gsem)
                    for t in range(GRP)
                ]
                puts = []
                for t in range(GRP):
                    gets[t].wait()
                    puts.append(pltpu.async_copy(
                        bufs[t], z_s.at[idst_v.at[b, t]], ssem, add=True))
                for p in puts:
                    p.wait()
            return 0
        lax.fori_loop(0, NG // 2, _group2, 0)

        plsc.subcore_barrier()

        def _rescale_chunk(q, _):
            row0 = s * RPT + q * RCH
            pltpu.sync_copy(z_s.at[pl.ds(row0, RCH)], buf_a)
            _scale_rows(buf_a, q)
            pltpu.sync_copy(buf_a, hs_hbm.at[k + 1, c, pl.ds(row0, RCH)])
            _scale_rows(buf_a, q)
            pltpu.sync_copy(buf_a, ytab_s.at[pl.ds(row0, RCH)])
            pltpu.sync_copy(buf_a, z_s.at[pl.ds(row0, RCH)])
            return 0
        lax.fori_loop(0, NRCH, _rescale_chunk, 0)
        return 0
    lax.fori_loop(0, K, _hop, 0)


@jax.jit
def _sc_propagate(feat_pad, src3, dst3):
    mesh = plsc.VectorSubcoreMesh(core_axis_name="c", subcore_axis_name="s")
    f = pl.kernel(
        _sc_body,
        out_type=(
            jax.ShapeDtypeStruct((K + 1, NC, NP, DH), jnp.float32),
        ),
        mesh=mesh,
        scratch_types=[
            pltpu.VMEM((RCH, DH), jnp.float32),
            pltpu.VMEM((RCH, DH), jnp.float32),
            pltpu.VMEM((2, GRP, CH), jnp.int32),
            pltpu.VMEM((2, GRP, CH), jnp.int32),
            pltpu.VMEM((NP,), jnp.float32),
            pltpu.VMEM((RPT,), jnp.float32),
            pltpu.VMEM((RPT,), jnp.float32),
            pltpu.SemaphoreType.DMA,
            pltpu.SemaphoreType.DMA,
            pltpu.SemaphoreType.DMA,
            pltpu.SemaphoreType.DMA,
            pltpu.VMEM_SHARED((NP, DH), jnp.float32),
            pltpu.VMEM_SHARED((NP, DH), jnp.float32),
            pltpu.VMEM_SHARED((NS, NP), jnp.float32),
        ],
        compiler_params=pltpu.CompilerParams(
            use_tc_tiling_on_sc=False, needs_layout_passes=False),
    )
    return f(feat_pad, src3, dst3)


def _mm_body(hs_ref, w_ref, b_ref, o_ref):
    tt = pl.program_id(1)

    @pl.when(tt == 0)
    def _():
        o_ref[...] = jnp.broadcast_to(b_ref[...], o_ref.shape)

    o_ref[...] += jnp.dot(hs_ref[0, 0], w_ref[0, 0],
                          preferred_element_type=jnp.float32)


@jax.jit
def _tc_linear(hs, W4, b2):
    BM = 1024
    return pl.pallas_call(
        _mm_body,
        grid=(NP // BM, (K + 1) * NC),
        in_specs=[
            pl.BlockSpec((1, 1, BM, DH), lambda r, t: (t // NC, t % NC, r, 0)),
            pl.BlockSpec((1, 1, DH, D), lambda r, t: (t // NC, t % NC, 0, 0)),
            pl.BlockSpec((1, D), lambda r, t: (0, 0)),
        ],
        out_specs=pl.BlockSpec((BM, D), lambda r, t: (r, 0)),
        out_shape=jax.ShapeDtypeStruct((NP, D), jnp.float32),
        compiler_params=pltpu.CompilerParams(
            dimension_semantics=("parallel", "arbitrary")),
    )(hs, W4, b2)


def kernel(feature, edge_index, W, b):
    src = edge_index[0].astype(jnp.int32)
    dst = edge_index[1].astype(jnp.int32)
    pad = EPAD - E
    fake = jnp.full((pad,), FAKE, jnp.int32)
    src3 = jnp.concatenate([src, fake]).reshape(NS, NCHUNK, CH)
    dst3 = jnp.concatenate([dst, fake]).reshape(NS, NCHUNK, CH)
    feat_pad = jnp.pad(feature, ((0, NP - N), (0, 0)))
    feat_cs = jnp.stack([feat_pad[:, :DH], feat_pad[:, DH:]])

    (hs,) = _sc_propagate(feat_cs, src3, dst3)
    out = _tc_linear(hs, W.reshape(K + 1, NC, DH, D), b.reshape(1, D))
    return out[:N]


# speedup vs baseline: 15.0540x; 1.0735x over previous
import jax
import jax.numpy as jnp
from jax import lax
from jax.experimental import pallas as pl
from jax.experimental.pallas import tpu as pltpu
from jax.experimental.pallas import tpu_sc as plsc

N = 10000
E = 320000
D = 128
K = 8
NC = 2
NS = 16
L = 16
DH = D // NC

NP = 10240
RPT = NP // NS
RCH = 128
NRCH = RPT // RCH
CH = 128
NCHUNK = 162
EPT = NCHUNK * CH
EPAD = NS * EPT
FAKE = NP - 1
NRING = 6
NBUF = 3
JJ = NCHUNK // NRING


def _rsqrt_newton(x):
    i = plsc.bitcast(x, jnp.int32)
    i = jnp.int32(0x5F3759DF) - (i >> 1)
    y = plsc.bitcast(i, jnp.float32)
    for _ in range(3):
        y = y * (1.5 - 0.5 * x * y * y)
    return y


def _sc_body(feat_hbm, srci_hbm, dsti_hbm, zs_hbm, dis_hbm,
             buf0, buf1, buf2, isrc_v, idst_v, deg_v, tmp_v, dis_v, disq_v,
             gsem0, gsem1, gsem2, ssem0, ssem1, ssem2,
             isem0, isem1, isem2, isem3, isem4, isem5,
             ytab_s, z_s, degp_s):
    c = lax.axis_index("c")
    s = lax.axis_index("s")
    bufs = [buf0, buf1, buf2]
    gsems = [gsem0, gsem1, gsem2]
    ssems = [ssem0, ssem1, ssem2]
    isems = [isem0, isem1, isem2, isem3, isem4, isem5]
    zero16 = jnp.zeros((L,), jnp.float32)
    one16 = jnp.ones((L,), jnp.float32)

    def _idx_copy(j, slot, dst_only=False):
        if not dst_only:
            pltpu.async_copy(srci_hbm.at[s, j], isrc_v.at[slot], isems[slot])
        pltpu.async_copy(dsti_hbm.at[s, j], idst_v.at[slot], isems[slot])

    def _idx_wait(slot, dst_only=False):
        if not dst_only:
            pltpu.make_async_copy(
                srci_hbm.at[s, 0], isrc_v.at[slot], isems[slot]).wait()
        pltpu.make_async_copy(
            dsti_hbm.at[s, 0], idst_v.at[slot], isems[slot]).wait()

    def _zero_deg(i, _):
        deg_v[pl.ds(i * L, L)] = zero16
        return 0
    lax.fori_loop(0, NP // L, _zero_deg, 0)

    for m in range(4):
        _idx_copy(m, m, dst_only=True)

    def _deg_outer(jj, _):
        for u in range(NRING):
            j = jj * NRING + u
            _idx_wait(u, dst_only=True)

            def _count(i, _):
                idx = idst_v[u, pl.ds(i * L, L)]
                plsc.addupdate_scatter(deg_v, [idx], one16)
                return 0
            lax.fori_loop(0, CH // L, _count, 0)

            if u < 2:
                _idx_copy(j + 4, (u + 4) % NRING, dst_only=True)
            else:
                @pl.when(jj < JJ - 1)
                def _():
                    _idx_copy(j + 4, (u + 4) % NRING, dst_only=True)
        return 0
    lax.fori_loop(0, JJ, _deg_outer, 0)

    pltpu.sync_copy(deg_v, degp_s.at[s])
    plsc.subcore_barrier()

    def _zero_dis(i, _):
        dis_v[pl.ds(i * L, L)] = zero16
        return 0
    lax.fori_loop(0, RPT // L, _zero_dis, 0)
    for r in range(NS):
        pltpu.sync_copy(degp_s.at[r, pl.ds(s * RPT, RPT)], tmp_v)

        def _acc(i, _):
            dis_v[pl.ds(i * L, L)] = (dis_v[pl.ds(i * L, L)]
                                      + tmp_v[pl.ds(i * L, L)])
            return 0
        lax.fori_loop(0, RPT // L, _acc, 0)

    def _dis_chunk(i, _):
        dv = _rsqrt_newton(dis_v[pl.ds(i * L, L)] + 1.0)
        dis_v[pl.ds(i * L, L)] = dv
        disq_v[pl.ds(i * L, L)] = dv * dv
        return 0
    lax.fori_loop(0, RPT // L, _dis_chunk, 0)

    @pl.when(c == 0)
    def _():
        pltpu.sync_copy(dis_v, dis_hbm.at[pl.ds(s * RPT, RPT)])

    def _scale_rows(buf, q, sc_v):
        def _grp(g, _):
            dv = sc_v[pl.ds(q * RCH + g * L, L)]
            for rr in range(L):
                sv = jnp.full((L,), dv[rr], jnp.float32)
                r = g * L + rr
                for jj in range(DH // L):
                    buf[r, pl.ds(jj * L, L)] = buf[r, pl.ds(jj * L, L)] * sv
            return 0
        lax.fori_loop(0, RCH // L, _grp, 0)

    def _init_chunk(q, _):
        row0 = s * RPT + q * RCH
        pltpu.sync_copy(
            feat_hbm.at[pl.ds(row0, RCH), pl.ds(c * DH, DH)], buf0)
        _scale_rows(buf0, q, dis_v)
        pltpu.sync_copy(buf0, ytab_s.at[pl.ds(row0, RCH)])
        pltpu.sync_copy(buf0, z_s.at[pl.ds(row0, RCH)])
        return 0
    lax.fori_loop(0, NRCH, _init_chunk, 0)

    def _hop(k, _):
        plsc.subcore_barrier()

        for m in range(4):
            _idx_copy(m, m)
        _idx_wait(0)
        pltpu.async_copy(ytab_s.at[isrc_v.at[0]], bufs[0], gsems[0])

        def _pipe(jj, _):
            for u in range(NRING):
                j = jj * NRING + u
                m = u % NBUF
                m1 = (u + 1) % NBUF
                slot1 = (u + 1) % NRING
                pltpu.make_async_copy(
                    ytab_s.at[isrc_v.at[u]], bufs[m], gsems[m]).wait()
                pltpu.async_copy(
                    bufs[m], z_s.at[idst_v.at[u]], ssems[m], add=True)
                if u >= 2:
                    pltpu.make_async_copy(
                        bufs[m1], z_s.at[idst_v.at[(u + 4) % NRING]],
                        ssems[m1]).wait()
                else:
                    @pl.when(jj > 0)
                    def _():
                        pltpu.make_async_copy(
                            bufs[m1], z_s.at[idst_v.at[(u + 4) % NRING]],
                            ssems[m1]).wait()
                if u < 2:
                    _idx_copy(j + 4, (u + 4) % NRING)
                else:
                    @pl.when(jj < JJ - 1)
                    def _():
                        _idx_copy(j + 4, (u + 4) % NRING)
                if u < NRING - 1:
                    _idx_wait(slot1)
                    pltpu.async_copy(
                        ytab_s.at[isrc_v.at[slot1]], bufs[m1], gsems[m1])
                else:
                    @pl.when(jj < JJ - 1)
                    def _():
                        _idx_wait(slot1)
                        pltpu.async_copy(
                            ytab_s.at[isrc_v.at[slot1]], bufs[m1], gsems[m1])
            return 0
        lax.fori_loop(0, JJ, _pipe, 0)

        pltpu.make_async_copy(
            bufs[(NCHUNK - 2) % NBUF],
            z_s.at[idst_v.at[(NCHUNK - 2) % NRING]],
            ssems[(NCHUNK - 2) % NBUF]).wait()
        pltpu.make_async_copy(
            bufs[(NCHUNK - 1) % NBUF],
            z_s.at[idst_v.at[(NCHUNK - 1) % NRING]],
            ssems[(NCHUNK - 1) % NBUF]).wait()

        plsc.subcore_barrier()

        def _rescale_chunk(q, _):
            row0 = s * RPT + q * RCH
            pltpu.sync_copy(z_s.at[pl.ds(row0, RCH)], buf0)
            pltpu.sync_copy(buf0, zs_hbm.at[k, c, pl.ds(row0, RCH)])

            @pl.when(k + 1 < K)
            def _():
                _scale_rows(buf0, q, disq_v)
                pltpu.sync_copy(buf0, ytab_s.at[pl.ds(row0, RCH)])
                pltpu.sync_copy(buf0, z_s.at[pl.ds(row0, RCH)])
            return 0
        lax.fori_loop(0, NRCH, _rescale_chunk, 0)
        return 0
    lax.fori_loop(0, K, _hop, 0)


@jax.jit
def _sc_propagate(feat_pad, src3, dst3):
    mesh = plsc.VectorSubcoreMesh(core_axis_name="c", subcore_axis_name="s")
    f = pl.kernel(
        _sc_body,
        out_type=(
            jax.ShapeDtypeStruct((K, NC, NP, DH), jnp.float32),
            jax.ShapeDtypeStruct((NP,), jnp.float32),
        ),
        mesh=mesh,
        scratch_types=[
            pltpu.VMEM((RCH, DH), jnp.float32),
            pltpu.VMEM((RCH, DH), jnp.float32),
            pltpu.VMEM((RCH, DH), jnp.float32),
            pltpu.VMEM((NRING, CH), jnp.int32),
            pltpu.VMEM((NRING, CH), jnp.int32),
            pltpu.VMEM((NP,), jnp.float32),
            pltpu.VMEM((RPT,), jnp.float32),
            pltpu.VMEM((RPT,), jnp.float32),
            pltpu.VMEM((RPT,), jnp.float32),
            pltpu.SemaphoreType.DMA,
            pltpu.SemaphoreType.DMA,
            pltpu.SemaphoreType.DMA,
            pltpu.SemaphoreType.DMA,
            pltpu.SemaphoreType.DMA,
            pltpu.SemaphoreType.DMA,
            pltpu.SemaphoreType.DMA,
            pltpu.SemaphoreType.DMA,
            pltpu.SemaphoreType.DMA,
            pltpu.SemaphoreType.DMA,
            pltpu.SemaphoreType.DMA,
            pltpu.SemaphoreType.DMA,
            pltpu.VMEM_SHARED((NP, DH), jnp.float32),
            pltpu.VMEM_SHARED((NP, DH), jnp.float32),
            pltpu.VMEM_SHARED((NS, NP), jnp.float32),
        ],
        compiler_params=pltpu.CompilerParams(
            use_tc_tiling_on_sc=False, needs_layout_passes=False),
    )
    return f(feat_pad, src3, dst3)


def _mm_body(zs_ref, feat_ref, w_ref, dis_ref, b_ref, o_ref):
    tt = pl.program_id(1)
    nz = K * NC

    @pl.when(tt == 0)
    def _():
        o_ref[...] = jnp.dot(zs_ref[0, 0], w_ref[0, 0],
                             preferred_element_type=jnp.float32)

    @pl.when(jnp.logical_and(tt > 0, tt < nz))
    def _():
        o_ref[...] += jnp.dot(zs_ref[0, 0], w_ref[0, 0],
                              preferred_element_type=jnp.float32)

    @pl.when(tt == nz)
    def _():
        o_ref[...] = (o_ref[...] * dis_ref[...] + b_ref[...]
                      + jnp.dot(feat_ref[:, :DH], w_ref[0, 0],
                                preferred_element_type=jnp.float32))

    @pl.when(tt == nz + 1)
    def _():
        o_ref[...] += jnp.dot(feat_ref[:, DH:], w_ref[0, 0],
                              preferred_element_type=jnp.float32)


@jax.jit
def _tc_linear(zs, feat_pad, W4, dis2, b2):
    BM = 1024
    nz = K * NC
    return pl.pallas_call(
        _mm_body,
        grid=(NP // BM, nz + 2),
        in_specs=[
            pl.BlockSpec(
                (1, 1, BM, DH),
                lambda r, t: (jnp.minimum(t // NC, K - 1), t % NC, r, 0)),
            pl.BlockSpec((BM, D), lambda r, t: (r, 0)),
            pl.BlockSpec(
                (1, 1, DH, D),
                lambda r, t: (jnp.where(t < nz, t // NC + 1, 0),
                              jnp.where(t < nz, t % NC, t - nz), 0, 0)),
            pl.BlockSpec((BM, 1), lambda r, t: (r, 0)),
            pl.BlockSpec((1, D), lambda r, t: (0, 0)),
        ],
        out_specs=pl.BlockSpec((BM, D), lambda r, t: (r, 0)),
        out_shape=jax.ShapeDtypeStruct((NP, D), jnp.float32),
        compiler_params=pltpu.CompilerParams(
            dimension_semantics=("parallel", "arbitrary")),
    )(zs, feat_pad, W4, dis2, b2)


def kernel(feature, edge_index, W, b):
    src = edge_index[0].astype(jnp.int32)
    dst = edge_index[1].astype(jnp.int32)
    pad = EPAD - E
    fake = jnp.full((pad,), FAKE, jnp.int32)
    src3 = jnp.concatenate([src, fake]).reshape(NS, NCHUNK, CH)
    dst3 = jnp.concatenate([dst, fake]).reshape(NS, NCHUNK, CH)
    feat_pad = jnp.pad(feature, ((0, NP - N), (0, 0)))

    zs, dis = _sc_propagate(feat_pad, src3, dst3)
    out = _tc_linear(zs, feat_pad, W.reshape(K + 1, NC, DH, D),
                     dis.reshape(NP, 1), b.reshape(1, D))
    return out[:N]
